# Initial kernel scaffold; baseline (speedup 1.0000x reference)
import jax
import jax.numpy as jnp
from jax.experimental import pallas as pl


def kernel(x, wide_table, deep_table, ln_gamma, ln_beta, W0, b0, W1, b1, W2, b2, W3, b3, W4, b4):
    raise NotImplementedError("write your pallas kernel here")


# speedup vs baseline: 1.1923x; 1.1923x over previous
import functools

import jax
import jax.numpy as jnp
from jax import lax
from jax.experimental import pallas as pl
from jax.experimental.pallas import tpu as pltpu
from jax.experimental.pallas import tpu_sc as plsc

B = 16384
F = 26
V = 1000000
D = 16

NC = 2
NS = 16
NW = NC * NS

IDX_PER_W = (B * F) // NW
ROWS_PER_W = B // NW
B_CHUNK = 64
IDX_CHUNK = B_CHUNK * F
NSUB = IDX_CHUNK // 128
N_CHUNKS = ROWS_PER_W // B_CHUNK


def _sc_gather_body(x_hbm, wide_hbm, deep_hbm, deep_out, wide_out,
                    idx_v, drows_v, wrows_v, wpart_v, sem):
    wid = lax.axis_index("s") * NC + lax.axis_index("c")
    rows_per_w = IDX_PER_W // 128

    pltpu.sync_copy(x_hbm.at[pl.ds(wid * rows_per_w, rows_per_w)], idx_v)

    def chunk_body(c, _):
        def issue(j, _):
            pltpu.async_copy(deep_hbm.at[idx_v.at[c * NSUB + j]],
                             drows_v.at[pl.ds(j * 128, 128)], sem)
            pltpu.async_copy(wide_hbm.at[idx_v.at[c * NSUB + j]],
                             wrows_v.at[pl.ds(j * 128, 128)], sem)
            return _
        lax.fori_loop(0, NSUB, issue, 0, unroll=False)

        def drain(j, _):
            pltpu.make_async_copy(deep_hbm.at[pl.ds(0, 128)],
                                  drows_v.at[pl.ds(0, 128)], sem).wait()
            return _
        lax.fori_loop(0, 2 * NSUB, drain, 0, unroll=False)

        pltpu.sync_copy(
            drows_v, deep_out.at[pl.ds(wid * IDX_PER_W + c * IDX_CHUNK,
                                       IDX_CHUNK)])

        def wacc(b, _):
            base = b * F
            acc = wrows_v[base]
            for f in range(1, F):
                acc = acc + wrows_v[base + f]
            wpart_v[b] = acc
            return _
        lax.fori_loop(0, B_CHUNK, wacc, 0, unroll=False)

        pltpu.sync_copy(
            wpart_v, wide_out.at[pl.ds(wid * ROWS_PER_W + c * B_CHUNK,
                                       B_CHUNK)])
        return _

    lax.fori_loop(0, N_CHUNKS, chunk_body, 0, unroll=False)


def _sc_gather(x_flat2d, wide_table, deep_table):
    mesh = plsc.VectorSubcoreMesh(core_axis_name="c", subcore_axis_name="s",
                                  num_cores=NC, num_subcores=NS)
    f = pl.kernel(
        _sc_gather_body,
        out_type=[
            jax.ShapeDtypeStruct((B * F, D), jnp.float32),
            jax.ShapeDtypeStruct((B, D), jnp.float32),
        ],
        mesh=mesh,
        scratch_types=[
            pltpu.VMEM((IDX_PER_W // 128, 128), jnp.int32),
            pltpu.VMEM((IDX_CHUNK, D), jnp.float32),
            pltpu.VMEM((IDX_CHUNK, D), jnp.float32),
            pltpu.VMEM((B_CHUNK, D), jnp.float32),
            pltpu.SemaphoreType.DMA,
        ],
        compiler_params=pltpu.CompilerParams(use_tc_tiling_on_sc=False),
    )
    return f(x_flat2d, wide_table, deep_table)


def _mlp_body(deep_ref, wpart_ref, gamma_ref, beta_ref,
              w0, b0, w1, b1, w2, b2, w3, b3, w4, b4, out_ref):
    h = deep_ref[...]
    mu = jnp.mean(h, axis=-1, keepdims=True)
    hc = h - mu
    var = jnp.mean(hc * hc, axis=-1, keepdims=True)
    h = hc * lax.rsqrt(var + 1e-5) * gamma_ref[...] + beta_ref[...]
    h = jnp.maximum(jnp.dot(h, w0[...], preferred_element_type=jnp.float32)
                    + b0[...], 0.0)
    h = jnp.maximum(jnp.dot(h, w1[...], preferred_element_type=jnp.float32)
                    + b1[...], 0.0)
    h = jnp.maximum(jnp.dot(h, w2[...], preferred_element_type=jnp.float32)
                    + b2[...], 0.0)
    h = jnp.maximum(jnp.dot(h, w3[...], preferred_element_type=jnp.float32)
                    + b3[...], 0.0)
    dnn = jnp.dot(h, w4[...], preferred_element_type=jnp.float32) + b4[...]
    wide = jnp.sum(wpart_ref[...], axis=-1, keepdims=True)
    out_ref[...] = jax.nn.sigmoid(dnn + wide)


def _mlp(deep_emb, wpart, ln_gamma, ln_beta, Ws, bs, block_b=1024):
    d_in = F * D
    full = lambda shape: pl.BlockSpec(shape, lambda i: (0, 0))
    in_specs = [
        pl.BlockSpec((block_b, d_in), lambda i: (i, 0)),
        pl.BlockSpec((block_b, D), lambda i: (i, 0)),
        full((1, d_in)),
        full((1, d_in)),
    ]
    args = [deep_emb, wpart, ln_gamma.reshape(1, d_in), ln_beta.reshape(1, d_in)]
    for w, b in zip(Ws, bs):
        in_specs.append(full(w.shape))
        in_specs.append(full((1, b.shape[0])))
        args.append(w)
        args.append(b.reshape(1, -1))
    return pl.pallas_call(
        _mlp_body,
        grid=(B // block_b,),
        in_specs=in_specs,
        out_specs=pl.BlockSpec((block_b, 1), lambda i: (i, 0)),
        out_shape=jax.ShapeDtypeStruct((B, 1), jnp.float32),
        compiler_params=pltpu.CompilerParams(
            dimension_semantics=("arbitrary",)),
    )(*args)


def kernel(x, wide_table, deep_table, ln_gamma, ln_beta,
           W0, b0, W1, b1, W2, b2, W3, b3, W4, b4):
    x_flat2d = x.reshape(B * F // 128, 128)
    deep_rows, wpart = _sc_gather(x_flat2d, wide_table, deep_table)
    deep_emb = deep_rows.reshape(B, F * D)
    return _mlp(deep_emb, wpart, ln_gamma, ln_beta,
                [W0, W1, W2, W3, W4], [b0, b1, b2, b3, b4])
